# Initial kernel scaffold; baseline (speedup 1.0000x reference)
import jax
import jax.numpy as jnp
from jax.experimental import pallas as pl


def kernel(features, labels):
    raise NotImplementedError("write your pallas kernel here")


# speedup vs baseline: 3.0436x; 3.0436x over previous
import functools

import jax
import jax.numpy as jnp
from jax.experimental import pallas as pl

TEMP = 0.07
K_HARD = 5
EPS = 1e-08


def _body(f_ref, fr_ref, labr_ref, labc_ref, out_ref, *, block_rows, batch):
    i = pl.program_id(0)

    f = f_ref[...]
    norms = jnp.sqrt(jnp.sum(f * f, axis=1, keepdims=True))
    fn = f / jnp.maximum(norms, 1e-12)
    fr = fr_ref[...]
    rnorms = jnp.sqrt(jnp.sum(fr * fr, axis=1, keepdims=True))
    rows = fr / jnp.maximum(rnorms, 1e-12)

    sim = jax.lax.dot_general(
        rows, fn, (((1,), (1,)), ((), ())), preferred_element_type=jnp.float32
    )
    sim = jnp.clip(sim, -10.0, 10.0) / TEMP

    labr = labr_ref[...]
    labc = labc_ref[...]
    col = jax.lax.broadcasted_iota(jnp.int32, (block_rows, batch), 1)
    rowg = jax.lax.broadcasted_iota(jnp.int32, (block_rows, batch), 0) + i * block_rows
    same = labr == labc
    eye = col == rowg
    pos_f = jnp.where(same & (~eye), 1.0, 0.0)
    neg_f = jnp.where(same, 0.0, 1.0)

    neg_scores = sim * neg_f
    num_neg = jnp.sum(neg_f, axis=1, keepdims=True)
    mx = jnp.max(sim, axis=1, keepdims=True)
    e = jnp.exp(sim - mx)
    exp_pos = jnp.sum(e * pos_f, axis=1, keepdims=True)
    s_all = jnp.sum(jnp.exp(neg_scores - mx), axis=1, keepdims=True)

    work = neg_scores
    tops = []
    for k in range(K_HARD):
        v = jnp.max(work, axis=1, keepdims=True)
        tops.append(v)
        if k < K_HARD - 1:
            idx = jnp.min(
                jnp.where(work == v, col, jnp.int32(2**30)), axis=1, keepdims=True
            )
            work = jnp.where(col == idx, -jnp.inf, work)

    actual_k = jnp.minimum(num_neg, float(K_HARD))
    ehn = []
    sum_hard = jnp.zeros_like(mx)
    for k in range(K_HARD):
        tv = jnp.where(float(k) < actual_k, tops[k], 0.0)
        eh = jnp.exp(tv - mx)
        ehn.append(eh)
        sum_hard = sum_hard + eh

    acc = jnp.zeros((1, 1), jnp.float32)
    for k in range(K_HARD):
        denom = exp_pos + sum_hard + s_all - ehn[k]
        lp = jnp.log(exp_pos / (denom + EPS) + EPS)
        acc = acc + jnp.sum(lp, axis=(0, 1), keepdims=True)

    @pl.when(i == 0)
    def _():
        out_ref[...] = jnp.zeros((1, 1), jnp.float32)

    out_ref[...] = out_ref[...] + acc


@jax.jit
def kernel(features, labels):
    batch, _ = features.shape
    block_rows = 512
    grid = (batch // block_rows,)

    labf = labels.astype(jnp.float32)
    labr = labf.reshape(batch, 1)
    labc = labf.reshape(1, batch)

    out = pl.pallas_call(
        functools.partial(_body, block_rows=block_rows, batch=batch),
        grid=grid,
        in_specs=[
            pl.BlockSpec(features.shape, lambda i: (0, 0)),
            pl.BlockSpec((block_rows, features.shape[1]), lambda i: (i, 0)),
            pl.BlockSpec((block_rows, 1), lambda i: (i, 0)),
            pl.BlockSpec((1, batch), lambda i: (0, 0)),
        ],
        out_specs=pl.BlockSpec((1, 1), lambda i: (0, 0)),
        out_shape=jax.ShapeDtypeStruct((1, 1), jnp.float32),
    )(features, features, labr, labc)

    return -out[0, 0] / (batch * K_HARD)


# speedup vs baseline: 3.4425x; 1.1311x over previous
import functools

import jax
import jax.numpy as jnp
from jax.experimental import pallas as pl

TEMP = 0.07
K_HARD = 5
EPS = 1e-08


def _body(f_ref, fr_ref, labr_ref, labc_ref, out_ref, *, block_rows, batch):
    i = pl.program_id(0)

    f = f_ref[...]
    norms = jnp.sqrt(jnp.sum(f * f, axis=1, keepdims=True))
    fn = f / jnp.maximum(norms, 1e-12)
    fr = fr_ref[...]
    rnorms = jnp.sqrt(jnp.sum(fr * fr, axis=1, keepdims=True))
    rows = fr / jnp.maximum(rnorms, 1e-12)

    sim = jax.lax.dot_general(
        rows, fn, (((1,), (1,)), ((), ())), preferred_element_type=jnp.float32
    )
    sim = jnp.clip(sim, -10.0, 10.0) / TEMP

    labr = labr_ref[...]
    labc = labc_ref[...]
    same = labr == labc

    num_neg = batch - jnp.sum(same.astype(jnp.float32), axis=1, keepdims=True)
    mx = jnp.max(sim, axis=1, keepdims=True)
    e = jnp.exp(sim - mx)
    emx = jnp.exp(-mx)

    col = jax.lax.broadcasted_iota(jnp.int32, (block_rows, batch), 1)
    rowg = jax.lax.broadcasted_iota(jnp.int32, (block_rows, batch), 0) + i * block_rows
    pos = same & (col != rowg)
    exp_pos = jnp.sum(jnp.where(pos, e, 0.0), axis=1, keepdims=True)

    en = jnp.where(same, emx, e)
    s_all = jnp.sum(en, axis=1, keepdims=True)

    work = en
    vals, cnts = [], []
    for k in range(K_HARD):
        v = jnp.max(work, axis=1, keepdims=True)
        eqm = work == v
        c = jnp.sum(jnp.where(eqm, 1.0, 0.0), axis=1, keepdims=True)
        vals.append(v)
        cnts.append(c)
        if k < K_HARD - 1:
            work = jnp.where(eqm, -jnp.inf, work)

    actual_k = jnp.minimum(num_neg, float(K_HARD))
    ehn = []
    sum_hard = jnp.zeros_like(mx)
    for k in range(K_HARD):
        cum = jnp.zeros_like(mx)
        slot_v = jnp.full_like(mx, -jnp.inf)
        for r in range(K_HARD):
            take = (cum <= float(k)) & (float(k) < cum + cnts[r])
            slot_v = jnp.where(take, vals[r], slot_v)
            cum = cum + cnts[r]
        eh = jnp.where(float(k) < actual_k, slot_v, emx)
        ehn.append(eh)
        sum_hard = sum_hard + eh

    acc = jnp.zeros((1, 1), jnp.float32)
    for k in range(K_HARD):
        denom = exp_pos + sum_hard + s_all - ehn[k]
        lp = jnp.log(exp_pos / (denom + EPS) + EPS)
        acc = acc + jnp.sum(lp, axis=(0, 1), keepdims=True)

    @pl.when(i == 0)
    def _():
        out_ref[...] = jnp.zeros((1, 1), jnp.float32)

    out_ref[...] = out_ref[...] + acc


@jax.jit
def kernel(features, labels):
    batch, _ = features.shape
    block_rows = 512
    grid = (batch // block_rows,)

    labf = labels.astype(jnp.float32)
    labr = labf.reshape(batch, 1)
    labc = labf.reshape(1, batch)

    out = pl.pallas_call(
        functools.partial(_body, block_rows=block_rows, batch=batch),
        grid=grid,
        in_specs=[
            pl.BlockSpec(features.shape, lambda i: (0, 0)),
            pl.BlockSpec((block_rows, features.shape[1]), lambda i: (i, 0)),
            pl.BlockSpec((block_rows, 1), lambda i: (i, 0)),
            pl.BlockSpec((1, batch), lambda i: (0, 0)),
        ],
        out_specs=pl.BlockSpec((1, 1), lambda i: (0, 0)),
        out_shape=jax.ShapeDtypeStruct((1, 1), jnp.float32),
    )(features, features, labr, labc)

    return -out[0, 0] / (batch * K_HARD)


# speedup vs baseline: 5.0532x; 1.4679x over previous
import functools

import jax
import jax.numpy as jnp
from jax.experimental import pallas as pl
from jax.experimental.pallas import tpu as pltpu

TEMP = 0.07
K_HARD = 5
EPS = 1e-08


def _merge_sorted(A, B, kmax):
    n, m = len(A), len(B)
    out = []
    for i in range(min(n + m, kmax)):
        cands = []
        if i < n:
            cands.append(A[i])
        if i < m:
            cands.append(B[i])
        for j in range(i):
            k2 = i - 1 - j
            if j < n and k2 < m:
                cands.append(jnp.minimum(A[j], B[k2]))
        c = cands[0]
        for x in cands[1:]:
            c = jnp.maximum(c, x)
        out.append(c)
    return out


def _body(f_ref, labr_ref, labc_ref, cid_ref, out_ref, fn_ref, *, block_rows, batch):
    i = pl.program_id(0)

    @pl.when(i == 0)
    def _():
        f = f_ref[...]
        norms = jnp.sqrt(jnp.sum(f * f, axis=1, keepdims=True))
        fn_ref[...] = f / jnp.maximum(norms, 1e-12)

    fn = fn_ref[...]
    rows = fn_ref[pl.ds(i * block_rows, block_rows), :]

    raw = jax.lax.dot_general(
        rows, fn, (((1,), (1,)), ((), ())), preferred_element_type=jnp.float32
    )

    labr = labr_ref[...]
    labc = labc_ref[...]
    cid = cid_ref[...]
    same = labr == labc

    mx = jnp.max(raw, axis=1, keepdims=True)
    inv_t = jnp.float32(1.0 / TEMP)
    e = jnp.exp((raw - mx) * inv_t)
    emx = jnp.exp(-mx * inv_t)

    rid = (
        jax.lax.broadcasted_iota(jnp.int32, (block_rows, 1), 0) + i * block_rows
    ).astype(jnp.float32)
    pos = same & (cid != rid)
    exp_pos = jnp.sum(jnp.where(pos, e, 0.0), axis=1, keepdims=True)

    en = jnp.where(same, emx, e)
    s_all = jnp.sum(en, axis=1, keepdims=True)

    lists = [en]
    w = batch
    while w > 128:
        h = w // 2
        lists = _merge_sorted(
            [a[:, :h] for a in lists], [a[:, h:] for a in lists], K_HARD
        )
        w = h
    cand = jnp.concatenate(lists, axis=1)

    work = cand
    vals, cnts = [], []
    for k in range(K_HARD):
        v = jnp.max(work, axis=1, keepdims=True)
        eqm = work == v
        c = jnp.sum(jnp.where(eqm, 1.0, 0.0), axis=1, keepdims=True)
        vals.append(v)
        cnts.append(c)
        if k < K_HARD - 1:
            work = jnp.where(eqm, -jnp.inf, work)

    ehn = []
    sum_hard = jnp.zeros_like(mx)
    for k in range(K_HARD):
        cum = jnp.zeros_like(mx)
        slot_v = jnp.zeros_like(mx)
        for r in range(K_HARD):
            take = (cum <= float(k)) & (float(k) < cum + cnts[r])
            slot_v = jnp.where(take, vals[r], slot_v)
            cum = cum + cnts[r]
        ehn.append(slot_v)
        sum_hard = sum_hard + slot_v

    acc = jnp.zeros((1, 1), jnp.float32)
    for k in range(K_HARD):
        denom = exp_pos + sum_hard + s_all - ehn[k]
        lp = jnp.log(exp_pos / (denom + EPS) + EPS)
        acc = acc + jnp.sum(lp, axis=(0, 1), keepdims=True)

    @pl.when(i == 0)
    def _():
        out_ref[...] = jnp.zeros((1, 1), jnp.float32)

    out_ref[...] = out_ref[...] + acc


@jax.jit
def kernel(features, labels):
    batch, dim = features.shape
    block_rows = 512
    grid = (batch // block_rows,)

    labf = labels.astype(jnp.float32)
    labr = labf.reshape(batch, 1)
    labc = labf.reshape(1, batch)
    cid = jnp.arange(batch, dtype=jnp.float32).reshape(1, batch)

    out = pl.pallas_call(
        functools.partial(_body, block_rows=block_rows, batch=batch),
        grid=grid,
        in_specs=[
            pl.BlockSpec(features.shape, lambda i: (0, 0)),
            pl.BlockSpec((block_rows, 1), lambda i: (i, 0)),
            pl.BlockSpec((1, batch), lambda i: (0, 0)),
            pl.BlockSpec((1, batch), lambda i: (0, 0)),
        ],
        out_specs=pl.BlockSpec((1, 1), lambda i: (0, 0)),
        out_shape=jax.ShapeDtypeStruct((1, 1), jnp.float32),
        scratch_shapes=[pltpu.VMEM((batch, dim), jnp.float32)],
    )(features, labr, labc, cid)

    return -out[0, 0] / (batch * K_HARD)


# speedup vs baseline: 6.0911x; 1.2054x over previous
import functools

import jax
import jax.numpy as jnp
from jax.experimental import pallas as pl
from jax.experimental.pallas import tpu as pltpu

TEMP = 0.07
K_HARD = 5
EPS = 1e-08


def _merge_sorted(A, B, kmax):
    n, m = len(A), len(B)
    out = []
    for i in range(min(n + m, kmax)):
        cands = []
        if i < n:
            cands.append(A[i])
        if i < m:
            cands.append(B[i])
        for j in range(i):
            k2 = i - 1 - j
            if j < n and k2 < m:
                cands.append(jnp.minimum(A[j], B[k2]))
        c = cands[0]
        for x in cands[1:]:
            c = jnp.maximum(c, x)
        out.append(c)
    return out


def _body(f_ref, labr_ref, labc_ref, cid_ref, out_ref, fn_ref, *, block_rows, batch):
    i = pl.program_id(0)

    @pl.when(i == 0)
    def _():
        f = f_ref[...]
        norms = jnp.sqrt(jnp.sum(f * f, axis=1, keepdims=True))
        fn_ref[...] = f / jnp.maximum(norms, 1e-12)

    fn = fn_ref[...]
    rows = fn_ref[pl.ds(i * block_rows, block_rows), :]

    raw = jax.lax.dot_general(
        rows, fn, (((1,), (1,)), ((), ())), preferred_element_type=jnp.float32
    )

    labr = labr_ref[...]
    labc = labc_ref[...]
    cid = cid_ref[...]
    same = labr == labc

    inv_t = jnp.float32(1.0 / TEMP)
    mx = jnp.max(raw, axis=1, keepdims=True)
    u = jnp.exp(raw * inv_t)
    emx = jnp.exp(-mx * inv_t)

    rid = (
        jax.lax.broadcasted_iota(jnp.int32, (block_rows, 1), 0) + i * block_rows
    ).astype(jnp.float32)
    pos = same & (cid != rid)
    exp_pos = emx * jnp.sum(jnp.where(pos, u, 0.0), axis=1, keepdims=True)

    en = jnp.where(same, 1.0, u)
    s_all = emx * jnp.sum(en, axis=1, keepdims=True)

    lists = [en]
    w = batch
    while w > 256:
        h = w // 2
        lists = _merge_sorted(
            [a[:, :h] for a in lists], [a[:, h:] for a in lists], K_HARD
        )
        w = h
    lists = [
        jnp.maximum(lists[i][:, :128], lists[K_HARD - 1 - i][:, 128:])
        for i in range(K_HARD)
    ]
    t = [jnp.transpose(a) for a in lists]
    for a, b in ((0, 1), (3, 4), (2, 4), (2, 3), (0, 3), (0, 2), (1, 4), (1, 3), (1, 2)):
        hi = jnp.maximum(t[a], t[b])
        lo = jnp.minimum(t[a], t[b])
        t[a], t[b] = hi, lo
    s = 128
    while s > 1:
        h = s // 2
        t = _merge_sorted([a[:h, :] for a in t], [a[h:, :] for a in t], K_HARD)
        s = h
    tops = [jnp.transpose(a) for a in t]

    ehn = [emx * v for v in tops]
    sum_hard = ehn[0] + ehn[1] + ehn[2] + ehn[3] + ehn[4]

    acc = jnp.zeros((1, 1), jnp.float32)
    base = exp_pos + sum_hard + s_all
    for r in range(K_HARD):
        denom = base - ehn[r]
        lp = jnp.log(exp_pos / (denom + EPS) + EPS)
        acc = acc + jnp.sum(lp, axis=(0, 1), keepdims=True)

    @pl.when(i == 0)
    def _():
        out_ref[...] = jnp.zeros((1, 1), jnp.float32)

    out_ref[...] = out_ref[...] + acc


@jax.jit
def kernel(features, labels):
    batch, dim = features.shape
    block_rows = 512
    grid = (batch // block_rows,)

    labf = labels.astype(jnp.float32)
    labr = labf.reshape(batch, 1)
    labc = labf.reshape(1, batch)
    cid = jnp.arange(batch, dtype=jnp.float32).reshape(1, batch)

    out = pl.pallas_call(
        functools.partial(_body, block_rows=block_rows, batch=batch),
        grid=grid,
        in_specs=[
            pl.BlockSpec(features.shape, lambda i: (0, 0)),
            pl.BlockSpec((block_rows, 1), lambda i: (i, 0)),
            pl.BlockSpec((1, batch), lambda i: (0, 0)),
            pl.BlockSpec((1, batch), lambda i: (0, 0)),
        ],
        out_specs=pl.BlockSpec((1, 1), lambda i: (0, 0)),
        out_shape=jax.ShapeDtypeStruct((1, 1), jnp.float32),
        scratch_shapes=[pltpu.VMEM((batch, dim), jnp.float32)],
    )(features, labr, labc, cid)

    return -out[0, 0] / (batch * K_HARD)
